# baseline (speedup 1.0000x reference)
import functools

import jax
import jax.numpy as jnp
from jax import lax
from jax.experimental import pallas as pl
from jax.experimental.pallas import tpu as pltpu
from jax.experimental.pallas import tpu_sc as plsc

V = 1_000_000
E = 64
CTX = 200
NC, NS, L = 2, 16, 16
NW = NC * NS
CPW = 8
ACTIVE = CTX // CPW


def _sc_gather_sum(idx_hbm, table_hbm, out_hbm, idx_v, rows_v, acc_v, sem):
    wid = lax.axis_index("s") * NC + lax.axis_index("c")
    base = wid * CPW
    zeros = jnp.zeros((L,), jnp.float32)
    for c in range(E // L):
        acc_v[pl.ds(c * L, L)] = zeros

    @pl.when(wid < ACTIVE)
    def _():
        pltpu.sync_copy(idx_hbm.at[pl.ds(base, CPW)], idx_v)
        pltpu.async_copy(table_hbm.at[idx_v], rows_v, sem).wait()
        for c in range(E // L):
            acc = rows_v[0, pl.ds(c * L, L)]
            for r in range(1, CPW):
                acc = acc + rows_v[r, pl.ds(c * L, L)]
            acc_v[pl.ds(c * L, L)] = acc

    pltpu.sync_copy(acc_v, out_hbm.at[wid])


@functools.cache
def _gather():
    return pl.kernel(
        _sc_gather_sum,
        out_type=jax.ShapeDtypeStruct((NW, E), jnp.float32),
        mesh=plsc.VectorSubcoreMesh(
            core_axis_name="c", subcore_axis_name="s", num_cores=NC, num_subcores=NS
        ),
        scratch_types=[
            pltpu.VMEM((CPW,), jnp.int32),
            pltpu.VMEM((CPW, E), jnp.float32),
            pltpu.VMEM((E,), jnp.float32),
            pltpu.SemaphoreType.DMA,
        ],
        compiler_params=pltpu.CompilerParams(use_tc_tiling_on_sc=False),
    )

BLK = 16_384
NB = (V + BLK - 1) // BLK
RC = 512


def _tc_matvec(part_ref, w_ref, b_ref, out_ref):
    emb = jnp.sum(part_ref[...], axis=0)

    def body(r, _):
        sl = pl.ds(r * RC, RC)
        w = w_ref[sl, :]
        out_ref[sl] = jnp.sum(w * emb[None, :], axis=1) + b_ref[sl]
        return 0

    lax.fori_loop(0, BLK // RC, body, 0)


_matvec = pl.pallas_call(
    _tc_matvec,
    grid=(NB,),
    in_specs=[
        pl.BlockSpec((NW, E), lambda i: (0, 0)),
        pl.BlockSpec((BLK, E), lambda i: (i, 0)),
        pl.BlockSpec((BLK,), lambda i: (i,)),
    ],
    out_specs=pl.BlockSpec((BLK,), lambda i: (i,)),
    out_shape=jax.ShapeDtypeStruct((V,), jnp.float32),
)


def kernel(inputs, emb_table, W, b):
    partials = _gather()(inputs, emb_table)
    return _matvec(partials, W, b)
